# Initial kernel scaffold; baseline (speedup 1.0000x reference)
import jax
import jax.numpy as jnp
from jax.experimental import pallas as pl


def kernel(x, edge_index, edge_type, goalVec, goalObjectsVec, W_rel_0, W_self_0, gate_w_0, gate_b_0, W_rel_1, W_self_1, gate_w_1, gate_b_1, W_rel_2, W_self_2, gate_w_2, gate_b_2, attn_W, attn_b, embed_W, embed_b, fc1_W, fc1_b, fc2_W, fc2_b, fc3_W, fc3_b, p1_W, p1_b, p2_W, p2_b, p3_W, p3_b, prelu_a):
    raise NotImplementedError("write your pallas kernel here")


# speedup vs baseline: 38.0244x; 38.0244x over previous
import functools

import jax
import jax.numpy as jnp
from jax import lax
from jax.experimental import pallas as pl
from jax.experimental.pallas import tpu as pltpu
from jax.experimental.pallas import tpu_sc as plsc

_N = 10000
_E = 320000
_H = 128
_NET = 4

_NW = 32
_EPW = _E // _NW
_CH = 125
_NCH = _EPW // _CH
_NPAD = 10240
_RPT = _NPAD // 16
_ZR = 32
_G = 16
_NG = _NCH // _G

_NB = 10
_BN = _N // _NB

_dot = functools.partial(jnp.dot, preferred_element_type=jnp.float32)


def _gidx_body(et_ref, src_ref, o_ref):
    o_ref[...] = et_ref[...] * _N + src_ref[...]


def _compute_gidx(et2d, src2d):
    return pl.pallas_call(
        _gidx_body,
        out_shape=jax.ShapeDtypeStruct((_E // _H, _H), jnp.int32),
    )(et2d, src2d)


def _ht0_body(h_ref, w_ref, o_ref):
    o_ref[0] = _dot(h_ref[...], w_ref[0])


def _compute_ht0(x, w_rel):
    return pl.pallas_call(
        _ht0_body,
        grid=(_NET, _NB),
        in_specs=[
            pl.BlockSpec((_BN, _H), lambda e, i: (i, 0)),
            pl.BlockSpec((1, _H, _H), lambda e, i: (e, 0, 0)),
        ],
        out_specs=pl.BlockSpec((1, _BN, _H), lambda e, i: (e, i, 0)),
        out_shape=jax.ShapeDtypeStruct((_NET, _N, _H), jnp.float32),
    )(x, w_rel)


def _upd_body(p_ref, h_ref, ws_ref, gw_ref, gb_ref, wr_ref, hn_ref, ht_ref):
    agg = p_ref[0] + p_ref[1]
    gate = jax.nn.sigmoid(_dot(agg, gw_ref[...]) + gb_ref[...])
    hn = jnp.maximum(gate * agg + _dot(h_ref[...], ws_ref[...]), 0.0)
    hn_ref[...] = hn
    for e in range(_NET):
        ht_ref[e] = _dot(hn, wr_ref[e])


def _layer_update(parts, h, w_self, gw, gb2d, w_rel_next):
    return pl.pallas_call(
        _upd_body,
        grid=(_NB,),
        in_specs=[
            pl.BlockSpec((2, _BN, _H), lambda i: (0, i, 0)),
            pl.BlockSpec((_BN, _H), lambda i: (i, 0)),
            pl.BlockSpec((_H, _H), lambda i: (0, 0)),
            pl.BlockSpec((_H, 1), lambda i: (0, 0)),
            pl.BlockSpec((1, 1), lambda i: (0, 0)),
            pl.BlockSpec((_NET, _H, _H), lambda i: (0, 0, 0)),
        ],
        out_specs=[
            pl.BlockSpec((_BN, _H), lambda i: (i, 0)),
            pl.BlockSpec((_NET, _BN, _H), lambda i: (0, i, 0)),
        ],
        out_shape=[
            jax.ShapeDtypeStruct((_N, _H), jnp.float32),
            jax.ShapeDtypeStruct((_NET, _N, _H), jnp.float32),
        ],
    )(parts, h, w_self, gw, gb2d, w_rel_next)


def _final_body(p_ref, h_ref, ws_ref, gw_ref, gb_ref,
                embW_ref, embB_ref, attnW_ref, attnb_ref,
                gv_ref, gov_ref, fc1W_ref, fc1b_ref, fc3W_ref, fc3b_ref,
                p2W_ref, p2b_ref, p3W_ref, p3b_ref, pa_ref,
                o_ref, acc_ref, ml_ref):
    i = pl.program_id(0)
    a2 = pa_ref[...]

    def prelu(v):
        return jnp.where(v > 0, v, a2 * v)

    @pl.when(i == 0)
    def _():
        ml_ref[0] = -jnp.inf
        ml_ref[1] = 0.0
        acc_ref[...] = jnp.zeros_like(acc_ref)

    agg = p_ref[0] + p_ref[1]
    gate = jax.nn.sigmoid(_dot(agg, gw_ref[...]) + gb_ref[...])
    h3 = jnp.maximum(gate * agg + _dot(h_ref[...], ws_ref[...]), 0.0)

    go = prelu(_dot(gov_ref[...], embW_ref[...]) + embB_ref[...])
    attnW = attnW_ref[...]
    c0 = _dot(go, attnW[_H:]) + attnb_ref[...]
    s = prelu(_dot(h3, attnW[:_H]) + c0)

    m_old = ml_ref[0]
    l_old = ml_ref[1]
    m_new = jnp.maximum(m_old, jnp.max(s))
    scale = jnp.exp(m_old - m_new)
    w = jnp.exp(s - m_new)
    ml_ref[0] = m_new
    ml_ref[1] = l_old * scale + jnp.sum(w)
    acc_ref[0:1] = acc_ref[0:1] * scale + jnp.sum(w * h3, axis=0, keepdims=True)

    @pl.when(i == _NB - 1)
    def _():
        scene = acc_ref[0:1] / ml_ref[1]
        ge = prelu(_dot(gv_ref[...], embW_ref[...]) + embB_ref[...])
        fc1W = fc1W_ref[...]
        h1 = prelu(_dot(scene, fc1W[:_H]) + _dot(ge, fc1W[_H:]) + fc1b_ref[...])
        t = prelu(_dot(h1, fc3W_ref[...]) + fc3b_ref[...])
        t = t - jnp.max(t)
        et = jnp.exp(t)
        tools = et / jnp.sum(et)
        ph = prelu(_dot(h1, p2W_ref[...]) + p2b_ref[...])
        pn = jax.nn.sigmoid(prelu(_dot(ph, p3W_ref[...]) + p3b_ref[...]))
        pad = jnp.zeros((1, _H - 11), jnp.float32)
        o_ref[...] = jnp.concatenate([(1.0 - pn) * tools, pn, pad], axis=1)


def _final(parts, h, w_self, gw, gb2d, embW, embB2d, attnW, attnb2d,
           gv2d, gov2d, fc1W, fc1b2d, fc3W, fc3b2d, p2W, p2b2d, p3W, p3b2d,
           pa2d):
    def full(shape):
        nd = len(shape)
        return pl.BlockSpec(shape, lambda i, _nd=nd: (0,) * _nd)
    return pl.pallas_call(
        _final_body,
        grid=(_NB,),
        in_specs=[
            pl.BlockSpec((2, _BN, _H), lambda i: (0, i, 0)),
            pl.BlockSpec((_BN, _H), lambda i: (i, 0)),
            full((_H, _H)),
            full((_H, 1)),
            full((1, 1)),
            full((300, _H)),
            full((1, _H)),
            full((2 * _H, 1)),
            full((1, 1)),
            full((1, 300)),
            full((1, 300)),
            full((2 * _H, _H)),
            full((1, _H)),
            full((_H, 10)),
            full((1, 10)),
            full((_H, _H)),
            full((1, _H)),
            full((_H, 1)),
            full((1, 1)),
            full((1, 1)),
        ],
        out_specs=pl.BlockSpec((1, _H), lambda i: (0, 0)),
        out_shape=jax.ShapeDtypeStruct((1, _H), jnp.float32),
        scratch_shapes=[
            pltpu.VMEM((8, _H), jnp.float32),
            pltpu.SMEM((2,), jnp.float32),
        ],
    )(parts, h, w_self, gw, gb2d, embW, embB2d, attnW, attnb2d,
      gv2d, gov2d, fc1W, fc1b2d, fc3W, fc3b2d, p2W, p2b2d, p3W, p3b2d, pa2d)


def _make_sc_agg():
    mesh = plsc.VectorSubcoreMesh(core_axis_name="c", subcore_axis_name="s")

    @functools.partial(
        pl.kernel,
        out_type=jax.ShapeDtypeStruct((2, _NPAD, _H), jnp.float32),
        mesh=mesh,
        scratch_types=[
            pltpu.VMEM((_G, _CH), jnp.int32),
            pltpu.VMEM((_G, _CH), jnp.int32),
            pltpu.VMEM((_G, _CH), jnp.int32),
            pltpu.VMEM((_G, _CH), jnp.int32),
            pltpu.VMEM((_CH, _H), jnp.float32),
            pltpu.VMEM((_CH, _H), jnp.float32),
            pltpu.VMEM((_ZR, _H), jnp.float32),
            pltpu.VMEM_SHARED((_NPAD, _H), jnp.float32),
            pltpu.SemaphoreType.DMA,
            pltpu.SemaphoreType.DMA,
            pltpu.SemaphoreType.DMA,
        ],
    )
    def sc_agg(ht_hbm, gidx_hbm, dst_hbm, out_hbm,
               gidx_a, gidx_b, dst_a, dst_b, rows_a, rows_b, zbuf, agg,
               sem_a, sem_b, sem_i):
        c = lax.axis_index("c")
        s = lax.axis_index("s")
        wid = s * 2 + c

        zero16 = jnp.zeros((16,), jnp.float32)

        def _zrow(i, carry):
            for jj in range(_H // 16):
                zbuf[i, pl.ds(jj * 16, 16)] = zero16
            return carry

        lax.fori_loop(0, _ZR, _zrow, 0)
        base = s * _RPT
        for k in range(_RPT // _ZR):
            pltpu.sync_copy(zbuf, agg.at[pl.ds(base + k * _ZR, _ZR)])

        pltpu.sync_copy(gidx_hbm.at[wid, pl.ds(0, _G)], gidx_a)
        pltpu.sync_copy(dst_hbm.at[wid, pl.ds(0, _G)], dst_a)
        plsc.subcore_barrier()

        def _issue(gref, j, buf, sem):
            pltpu.make_async_copy(ht_hbm.at[gref.at[j]], buf, sem).start()

        def _wait(buf, sem):
            pltpu.make_async_copy(ht_hbm.at[gidx_a.at[0]], buf, sem).wait()

        def _scat(dref, j, buf):
            pltpu.sync_copy(buf, agg.at[dref.at[j]], add=True)

        _issue(gidx_a, 0, rows_a, sem_a)

        ibufs = (gidx_a, dst_a), (gidx_b, dst_b)
        for g in range(_NG):
            gcur, dcur = ibufs[g % 2]
            if g + 1 < _NG:
                gnx, dnx = ibufs[(g + 1) % 2]
                pltpu.make_async_copy(
                    gidx_hbm.at[wid, pl.ds((g + 1) * _G, _G)], gnx, sem_i).start()
                pltpu.make_async_copy(
                    dst_hbm.at[wid, pl.ds((g + 1) * _G, _G)], dnx, sem_i).start()

            def _body(it, carry, gcur=gcur, dcur=dcur):
                j = 2 * it
                _issue(gcur, j + 1, rows_b, sem_b)
                _wait(rows_a, sem_a)
                _scat(dcur, j, rows_a)

                @pl.when(j + 2 < _G)
                def _():
                    _issue(gcur, j + 2, rows_a, sem_a)

                _wait(rows_b, sem_b)
                _scat(dcur, j + 1, rows_b)
                return carry

            lax.fori_loop(0, _G // 2, _body, 0)
            if g + 1 < _NG:
                pltpu.make_async_copy(
                    gidx_hbm.at[wid, pl.ds(0, _G)], gnx, sem_i).wait()
                pltpu.make_async_copy(
                    dst_hbm.at[wid, pl.ds(0, _G)], dnx, sem_i).wait()
                _issue(gnx, 0, rows_a, sem_a)

        plsc.subcore_barrier()

        for k in range(_RPT // _ZR):
            pltpu.sync_copy(agg.at[pl.ds(base + k * _ZR, _ZR)],
                            out_hbm.at[c, pl.ds(base + k * _ZR, _ZR)])

    return sc_agg


_sc_agg = _make_sc_agg()


def kernel(x, edge_index, edge_type, goalVec, goalObjectsVec,
           W_rel_0, W_self_0, gate_w_0, gate_b_0,
           W_rel_1, W_self_1, gate_w_1, gate_b_1,
           W_rel_2, W_self_2, gate_w_2, gate_b_2,
           attn_W, attn_b, embed_W, embed_b,
           fc1_W, fc1_b, fc2_W, fc2_b, fc3_W, fc3_b,
           p1_W, p1_b, p2_W, p2_b, p3_W, p3_b, prelu_a):
    del fc2_W, fc2_b, p1_W, p1_b

    src2d = edge_index[0].reshape(_E // _H, _H)
    et2d = edge_type.reshape(_E // _H, _H)
    gidx_r = _compute_gidx(et2d, src2d).reshape(_NW, _NCH, _CH)
    dst_r = edge_index[1].reshape(_NW, _NCH, _CH)

    as2 = lambda v: v.reshape(1, -1)
    gb0, gb1, gb2 = as2(gate_b_0), as2(gate_b_1), as2(gate_b_2)

    ht = _compute_ht0(x, W_rel_0).reshape(_NET * _N, _H)
    parts = _sc_agg(ht, gidx_r, dst_r)
    h, ht = _layer_update(parts, x, W_self_0, gate_w_0, gb0, W_rel_1)
    parts = _sc_agg(ht.reshape(_NET * _N, _H), gidx_r, dst_r)
    h, ht = _layer_update(parts, h, W_self_1, gate_w_1, gb1, W_rel_2)
    parts = _sc_agg(ht.reshape(_NET * _N, _H), gidx_r, dst_r)

    out2d = _final(parts, h, W_self_2, gate_w_2, gb2,
                   embed_W, as2(embed_b), attn_W, as2(attn_b),
                   as2(goalVec), as2(goalObjectsVec),
                   fc1_W, as2(fc1_b), fc3_W, as2(fc3_b),
                   p2_W, as2(p2_b), p3_W, as2(p3_b),
                   jnp.asarray(prelu_a, jnp.float32).reshape(1, 1))
    return out2d[0, :11]
